# baseline (speedup 1.0000x reference)
import functools

import numpy as np
import jax
import jax.numpy as jnp
from jax import lax
from jax.experimental import pallas as pl
from jax.experimental.pallas import tpu as pltpu
from jax.experimental.pallas import tpu_sc as plsc

N = 64
NUM_WORKERS = 32
TOTAL_ROWS = 16 * 512
ROWS_PER_WORKER = TOTAL_ROWS // NUM_WORKERS
NEG = -3.0e38


def _valid_mask_np():
    i = np.arange(N)[:, None]
    j = np.arange(N)[None, :]
    d = j - i
    m = (((d >= 0) & (d <= 15))
         | ((d >= 17) & (d <= 31) & (d % 2 == 1) & (i % 2 == 0))
         | ((d >= 35) & (d <= 63) & (d % 4 == 3) & (i % 4 == 0)))
    return m.astype(np.float32)


_GATHER_DNUMS = lax.GatherDimensionNumbers(
    offset_dims=(), collapsed_slice_dims=(0,), start_index_map=(0,))


def _splat(vec, r):
    idx = jnp.full((16,), r, jnp.int32)
    return lax.gather(vec, idx[:, None], _GATHER_DNUMS, (1,),
                      mode=lax.GatherScatterMode.PROMISE_IN_BOUNDS)


def _sc_band_max(x_hbm, mask_hbm, out_hbm, xbuf, maskbuf, tile0):
    wid = lax.axis_index("s") * 2 + lax.axis_index("c")
    base = wid * ROWS_PER_WORKER
    pltpu.sync_copy(x_hbm.at[pl.ds(base, ROWS_PER_WORKER)], xbuf)
    pltpu.sync_copy(mask_hbm, maskbuf)

    lane = lax.iota(jnp.int32, 16)

    def compute_tile(t, tilebuf):
        xb = [xbuf[t, pl.ds(c * 16, 16)] for c in range(4)]
        M = tuple(jnp.full((16,), NEG, jnp.float32) for _ in range(4))
        for b in (3, 2, 1, 0):
            def row_body(k, M, b=b):
                r = 15 - k
                i = b * 16 + r
                sp = _splat(xb[b], r)
                Mn = [jnp.maximum(sp, M[c]) for c in range(4)]
                Mn[b] = jnp.where(lane == r, sp, Mn[b])
                for c in range(4):
                    mv = maskbuf[i, pl.ds(c * 16, 16)]
                    tilebuf[i, pl.ds(c * 16, 16)] = Mn[c] * mv
                return tuple(Mn)
            M = lax.fori_loop(0, 16, row_body, M)

    def tile_loop(t, carry):
        compute_tile(t, tile0)
        pltpu.sync_copy(tile0, out_hbm.at[base + t])
        return carry

    lax.fori_loop(0, ROWS_PER_WORKER, tile_loop, 0)


@jax.jit
def _run(xf, mask):
    mesh = plsc.VectorSubcoreMesh(core_axis_name="c", subcore_axis_name="s")
    f = pl.kernel(
        _sc_band_max,
        mesh=mesh,
        out_type=jax.ShapeDtypeStruct((TOTAL_ROWS, N, N), jnp.float32),
        scratch_types=[
            pltpu.VMEM((ROWS_PER_WORKER, N), jnp.float32),
            pltpu.VMEM((N, N), jnp.float32),
            pltpu.VMEM((N, N), jnp.float32),
        ],
    )
    return f(xf, mask)


def kernel(x):
    B, D, n = x.shape
    xf = x.reshape(B * D, n)
    mask = jnp.asarray(_valid_mask_np())
    out = _run(xf, mask)
    return out.reshape(B, D, n, n)


# speedup vs baseline: 2.3951x; 2.3951x over previous
import functools

import numpy as np
import jax
import jax.numpy as jnp
from jax import lax
from jax.experimental import pallas as pl
from jax.experimental.pallas import tpu as pltpu
from jax.experimental.pallas import tpu_sc as plsc

N = 64
NUM_WORKERS = 32
TOTAL_ROWS = 16 * 512
ROWS_PER_WORKER = TOTAL_ROWS // NUM_WORKERS


def _valid_mask_np():
    i = np.arange(N)[:, None]
    j = np.arange(N)[None, :]
    d = j - i
    m = (((d >= 0) & (d <= 15))
         | ((d >= 17) & (d <= 31) & (d % 2 == 1) & (i % 2 == 0))
         | ((d >= 35) & (d <= 63) & (d % 4 == 3) & (i % 4 == 0)))
    return m.astype(np.float32)


_MASK_NP = _valid_mask_np()
_WRITE = [[c for c in range(4) if _MASK_NP[i, c * 16:(c + 1) * 16].any()]
          for i in range(N)]

_GATHER_DNUMS = lax.GatherDimensionNumbers(
    offset_dims=(), collapsed_slice_dims=(0,), start_index_map=(0,))


def _splat(vec, r, lane):
    idx = ((lane & 0) + r)[:, None]
    return lax.gather(vec, idx, _GATHER_DNUMS, (1,),
                      mode=lax.GatherScatterMode.PROMISE_IN_BOUNDS)


def _sc_band_max(x_hbm, mask_hbm, out_hbm,
                 xbuf, maskbuf, tile0, tile1, sem0, sem1):
    wid = lax.axis_index("s") * 2 + lax.axis_index("c")
    base = wid * ROWS_PER_WORKER
    pltpu.sync_copy(x_hbm.at[pl.ds(base, ROWS_PER_WORKER)], xbuf)
    pltpu.sync_copy(mask_hbm, maskbuf)

    lane = lax.iota(jnp.int32, 16)
    zero = (lane & 0).astype(jnp.float32)

    def zero_body(i, carry):
        for c in range(4):
            tile0[i, pl.ds(c * 16, 16)] = zero
            tile1[i, pl.ds(c * 16, 16)] = zero
        return carry

    lax.fori_loop(0, N, zero_body, 0)

    def compute_tile(t, tilebuf):
        xb = [xbuf[t, pl.ds(c * 16, 16)] for c in range(4)]
        M = [None] * 4
        for b in (3, 2, 1, 0):
            for r in range(15, -1, -1):
                i = 16 * b + r
                sp = _splat(xb[b], r, lane)
                for c in range(b, 4):
                    M[c] = sp if M[c] is None else jnp.maximum(sp, M[c])
                if r != 15:
                    M[b] = jnp.where(lane == r, sp, M[b])
                for c in _WRITE[i]:
                    mv = maskbuf[i, pl.ds(c * 16, 16)]
                    tilebuf[i, pl.ds(c * 16, 16)] = M[c] * mv

    bufs = ((tile0, sem0), (tile1, sem1))

    def pair_body(t2, carry):
        for phase, (buf, sem) in enumerate(bufs):
            t = 2 * t2 + phase
            cp = pltpu.make_async_copy(buf, out_hbm.at[base + t], sem)

            @pl.when(t2 > 0)
            def _():
                cp.wait()

            compute_tile(t, buf)
            cp.start()
        return carry

    lax.fori_loop(0, ROWS_PER_WORKER // 2, pair_body, 0)
    for phase, (buf, sem) in enumerate(bufs):
        pltpu.make_async_copy(
            buf, out_hbm.at[base + ROWS_PER_WORKER - 2 + phase], sem).wait()


@jax.jit
def _run(xf, mask):
    mesh = plsc.VectorSubcoreMesh(core_axis_name="c", subcore_axis_name="s")
    f = pl.kernel(
        _sc_band_max,
        mesh=mesh,
        out_type=jax.ShapeDtypeStruct((TOTAL_ROWS, N, N), jnp.float32),
        scratch_types=[
            pltpu.VMEM((ROWS_PER_WORKER, N), jnp.float32),
            pltpu.VMEM((N, N), jnp.float32),
            pltpu.VMEM((N, N), jnp.float32),
            pltpu.VMEM((N, N), jnp.float32),
            pltpu.SemaphoreType.DMA,
            pltpu.SemaphoreType.DMA,
        ],
    )
    return f(xf, mask)


def kernel(x):
    B, D, n = x.shape
    xf = x.reshape(B * D, n)
    mask = jnp.asarray(_MASK_NP)
    out = _run(xf, mask)
    return out.reshape(B, D, n, n)
